# baseline (speedup 1.0000x reference)
import functools

import jax
import jax.numpy as jnp
from jax import lax
from jax.experimental import pallas as pl
from jax.experimental.pallas import tpu as pltpu
from jax.experimental.pallas import tpu_sc as plsc

N = 50000
C = 512
G = 128
NC = 2
NS = 16
NW = NC * NS
GPW = G // NW
CH = 32
CHS = 5
NSL = C // 16


def _rsqrt_newton(x):
    i = plsc.bitcast(x, jnp.int32)
    y = plsc.bitcast(jnp.int32(0x5F3759DF) - (i >> 1), jnp.float32)
    for _ in range(3):
        y = y * (1.5 - 0.5 * x * y * y)
    return y


def _sc_body(x_hbm, st_hbm, w_hbm, bias_hbm, ms_hbm, out_hbm,
             st_v, bufa, bufb, bufc, obufa, obufb, acc1, acc2, av, bv, wv,
             biasv, msv, sema, semb, semc, osema, osemb):
    wid = lax.axis_index("s") * NC + lax.axis_index("c")

    pltpu.sync_copy(w_hbm, wv)
    pltpu.sync_copy(bias_hbm, biasv)
    pltpu.sync_copy(ms_hbm, msv)

    def start_in(base, buf, sem):
        pltpu.make_async_copy(
            x_hbm.at[pl.ds(base * C, CH * C)], buf, sem).start()

    def wait_in(buf, sem):
        pltpu.make_async_copy(
            x_hbm.at[pl.ds(0, CH * C)], buf, sem).wait()

    def start_out(obuf, base, osem):
        pltpu.make_async_copy(
            obuf, out_hbm.at[pl.ds(base * C, CH * C)], osem).start()

    def wait_out(obuf, osem):
        pltpu.make_async_copy(
            obuf, out_hbm.at[pl.ds(0, CH * C)], osem).wait()

    pltpu.sync_copy(st_hbm.at[pl.ds(16 * wid, 16)], st_v)
    s16 = st_v[...]

    for kk in range(GPW):
        r0 = s16[kk]
        r1 = s16[kk + 1]
        nrows = r1 - r0

        def zero_j(j, _):
            acc1[pl.ds(16 * j, 16)] = jnp.zeros((16,), jnp.float32)
            acc2[pl.ds(16 * j, 16)] = jnp.zeros((16,), jnp.float32)
            return 0

        lax.fori_loop(0, NSL, zero_j, 0)

        nfull = nrows >> CHS
        rem = nrows - (nfull << CHS)

        def acc_chunk(buf):
            def accj(j, _):
                off = 16 * j
                a1 = acc1[pl.ds(off, 16)]
                a2 = acc2[pl.ds(off, 16)]
                for r in range(CH):
                    x = buf[pl.ds(r * C + off, 16)]
                    a1 = a1 + x
                    a2 = a2 + x * x
                acc1[pl.ds(off, 16)] = a1
                acc2[pl.ds(off, 16)] = a2
                return 0

            lax.fori_loop(0, NSL, accj, 0)

        bufs = (bufa, bufb, bufc)
        sems = (sema, semb, semc)
        ncha = nfull + jnp.where(rem > 0, 1, 0)

        def a_base(ci):
            return jnp.minimum(r0 + ci * CH, N - CH)

        def acc_tail(buf):
            lo = r0 + nfull * CH
            shift = lo - a_base(nfull)

            def tail_r(r, _):
                ok = jnp.logical_and(r >= shift, r < shift + rem)
                fm = jnp.full((16,), jnp.where(ok, 1.0, 0.0),
                              dtype=jnp.float32)

                def tail_j(j, _):
                    sl = pl.ds(16 * j, 16)
                    x = buf[pl.ds(r * C + 16 * j, 16)] * fm
                    acc1[sl] = acc1[sl] + x
                    acc2[sl] = acc2[sl] + x * x
                    return 0

                lax.fori_loop(0, NSL, tail_j, 0)
                return 0

            lax.fori_loop(0, CH, tail_r, 0)

        @pl.when(ncha > 0)
        def _():
            start_in(a_base(0), bufa, sema)

            @pl.when(ncha > 1)
            def _():
                start_in(a_base(1), bufb, semb)

            def body(ci, p):
                for q in range(3):
                    def br(q=q):
                        wait_in(bufs[q], sems[q])

                        @pl.when(ci + 2 < ncha)
                        def _():
                            start_in(a_base(ci + 2),
                                     bufs[(q + 2) % 3], sems[(q + 2) % 3])

                        @pl.when(ci < nfull)
                        def _():
                            acc_chunk(bufs[q])

                        @pl.when(ci >= nfull)
                        def _():
                            acc_tail(bufs[q])
                    pl.when(p == q)(br)
                return jnp.where(p == 2, 0, p + 1)

            lax.fori_loop(0, ncha, body, jnp.int32(0))

        nch = (nrows + CH - 1) >> CHS
        big = nrows >= CH

        @pl.when(big)
        def _():
            start_in(r0, bufa, sema)

            @pl.when(nch > 1)
            def _():
                start_in(jnp.minimum(r0 + CH, r1 - CH), bufb, semb)

        nv = jnp.full((16,), nrows, dtype=jnp.float32)
        inv_n = 1.0 / jnp.maximum(nv, 1.0)

        def coeff_j(j, _):
            sl = pl.ds(16 * j, 16)
            m = acc1[sl] * inv_n
            ms = m * msv[sl]
            var = acc2[sl] * inv_n - 2.0 * ms * m + ms * ms
            rstd = _rsqrt_newton(var + 1e-6)
            a = wv[sl] * rstd
            av[sl] = a
            bv[sl] = biasv[sl] - a * ms
            return 0

        lax.fori_loop(0, NSL, coeff_j, 0)

        def apply_chunk(buf, obuf):
            def appj(j, _):
                off = 16 * j
                a = av[pl.ds(off, 16)]
                b = bv[pl.ds(off, 16)]
                for r in range(CH):
                    obuf[pl.ds(r * C + off, 16)] = (
                        buf[pl.ds(r * C + off, 16)] * a + b)
                return 0

            lax.fori_loop(0, NSL, appj, 0)

        @pl.when(big)
        def _():
            def body(ci, p):
                base = jnp.minimum(r0 + ci * CH, r1 - CH)
                nbase2 = jnp.minimum(r0 + (ci + 2) * CH, r1 - CH)
                even = (ci & 1) == 0
                for q in range(3):
                    def br(q=q):
                        wait_in(bufs[q], sems[q])

                        @pl.when(ci + 2 < nch)
                        def _():
                            start_in(nbase2, bufs[(q + 2) % 3],
                                     sems[(q + 2) % 3])

                        @pl.when(even)
                        def _():
                            @pl.when(ci >= 2)
                            def _():
                                wait_out(obufa, osema)
                            apply_chunk(bufs[q], obufa)
                            start_out(obufa, base, osema)

                        @pl.when(jnp.logical_not(even))
                        def _():
                            @pl.when(ci >= 2)
                            def _():
                                wait_out(obufb, osemb)
                            apply_chunk(bufs[q], obufb)
                            start_out(obufb, base, osemb)
                    pl.when(p == q)(br)
                return jnp.where(p == 2, 0, p + 1)

            lax.fori_loop(0, nch, body, jnp.int32(0))

            last_even = ((nch - 1) & 1) == 0

            @pl.when(last_even)
            def _():
                wait_out(obufa, osema)

            @pl.when(jnp.logical_not(last_even))
            def _():
                wait_out(obufb, osemb)

            @pl.when(jnp.logical_and(nch >= 2, last_even))
            def _():
                wait_out(obufb, osemb)

            @pl.when(jnp.logical_and(nch >= 2, jnp.logical_not(last_even)))
            def _():
                wait_out(obufa, osema)

        @pl.when(jnp.logical_and(nrows > 0, jnp.logical_not(big)))
        def _():
            def row_b(r, _):
                row = r0 + r
                pltpu.sync_copy(x_hbm.at[pl.ds(row * C, C)],
                                bufa.at[pl.ds(0, C)])

                def row_j(j, _):
                    obufa[pl.ds(16 * j, 16)] = (
                        bufa[pl.ds(16 * j, 16)] * av[pl.ds(16 * j, 16)]
                        + bv[pl.ds(16 * j, 16)])
                    return 0

                lax.fori_loop(0, NSL, row_j, 0)
                pltpu.sync_copy(obufa.at[pl.ds(0, C)],
                                out_hbm.at[pl.ds(row * C, C)])
                return 0

            lax.fori_loop(0, nrows, row_b, 0)


@functools.partial(jax.jit, static_argnums=(2,))
def _graph_norm(tensor, batch, num_graphs, weight, bias, mean_scale):
    del num_graphs
    x = tensor.reshape(N * C)
    bi = batch.astype(jnp.int32)
    starts = jnp.searchsorted(bi, jnp.arange(G + 1, dtype=jnp.int32),
                              method="compare_all")
    idx = jnp.clip(GPW * jnp.arange(NW, dtype=jnp.int32)[:, None]
                   + jnp.arange(16, dtype=jnp.int32)[None, :], 0, G)
    st16 = starts[idx].astype(jnp.int32).reshape(NW * 16)

    mesh = plsc.VectorSubcoreMesh(core_axis_name="c", subcore_axis_name="s",
                                  num_cores=NC, num_subcores=NS)
    run = pl.kernel(
        _sc_body,
        out_type=jax.ShapeDtypeStruct((N * C,), jnp.float32),
        mesh=mesh,
        scratch_types=[
            pltpu.VMEM((16,), jnp.int32),
            pltpu.VMEM((CH * C,), jnp.float32),
            pltpu.VMEM((CH * C,), jnp.float32),
            pltpu.VMEM((CH * C,), jnp.float32),
            pltpu.VMEM((CH * C,), jnp.float32),
            pltpu.VMEM((CH * C,), jnp.float32),
            pltpu.VMEM((C,), jnp.float32),
            pltpu.VMEM((C,), jnp.float32),
            pltpu.VMEM((C,), jnp.float32),
            pltpu.VMEM((C,), jnp.float32),
            pltpu.VMEM((C,), jnp.float32),
            pltpu.VMEM((C,), jnp.float32),
            pltpu.VMEM((C,), jnp.float32),
            pltpu.SemaphoreType.DMA,
            pltpu.SemaphoreType.DMA,
            pltpu.SemaphoreType.DMA,
            pltpu.SemaphoreType.DMA,
            pltpu.SemaphoreType.DMA,
        ],
        compiler_params=pltpu.CompilerParams(needs_layout_passes=False),
    )
    out = run(x, st16, weight, bias, mean_scale)
    return out.reshape(N, C, 1)


def kernel(tensor, batch, num_graphs, weight, bias, mean_scale):
    return _graph_norm(tensor, batch, G, weight, bias, mean_scale)
